# Initial kernel scaffold; baseline (speedup 1.0000x reference)
import jax
import jax.numpy as jnp
from jax.experimental import pallas as pl


def kernel(x_drug, x_protein, x_cell, edge_index_dd, edge_attr_dd, edge_src_pd, edge_dst_pd, edge_src_pc, edge_dst_pc, drug1, drug2, cell, drug_emb, protein_emb, cell_emb, W_dd, att_src_dd, att_dst_dd, lin_edge_dd, att_edge_dd, bias_dd, W_pd, att_src_pd, att_dst_pd, bias_pd, W_pc, att_src_pc, att_dst_pc, bias_pc, W1, b1, W2, b2, W3, b3):
    raise NotImplementedError("write your pallas kernel here")


# speedup vs baseline: 13.5094x; 13.5094x over previous
import functools

import jax
import jax.numpy as jnp
from jax import lax
from jax.experimental import pallas as pl
from jax.experimental.pallas import tpu as pltpu
from jax.experimental.pallas import tpu_sc as plsc

H = 128
NDP = 10240
NCP = 1024
B = 4096
EDD, EPD, EPC = 160000, 320000, 64000
NT = 16
K = 80
KL = 64
KQ = 64


def _k1a_body(xd, xp, wdd, wpd, wpc, asdd, addd, aspd, adpd, aspc,
              hdd_o, hpd_o, hpc_o, ssdd_o, sddd_o, sspd_o, sdpd_o, sspc_o):
    d = xd[...]
    p = xp[...]
    m1 = jnp.dot(d, wdd[...], preferred_element_type=jnp.float32)
    hdd_o[...] = m1
    ssdd_o[...] = (m1 * asdd[...]).sum(axis=1, keepdims=True)
    sddd_o[...] = (m1 * addd[...]).sum(axis=1, keepdims=True)
    m2 = jnp.dot(p, wpd[...], preferred_element_type=jnp.float32)
    hpd_o[...] = m2
    sspd_o[...] = (m2 * aspd[...]).sum(axis=1, keepdims=True)
    m3 = jnp.dot(d, wpd[...], preferred_element_type=jnp.float32)
    sdpd_o[...] = (m3 * adpd[...]).sum(axis=1, keepdims=True)
    m4 = jnp.dot(p, wpc[...], preferred_element_type=jnp.float32)
    hpc_o[...] = m4
    sspc_o[...] = (m4 * aspc[...]).sum(axis=1, keepdims=True)


def _k1a(xd, xp, wdd, wpd, wpc, asdd, addd, aspd, adpd, aspc):
    nblk = NDP // 512
    row = pl.BlockSpec((512, H), lambda i: (i, 0))
    full = pl.BlockSpec((H, H), lambda i: (0, 0))
    vec = pl.BlockSpec((1, H), lambda i: (0, 0))
    col = pl.BlockSpec((512, 1), lambda i: (i, 0))
    fmat = jax.ShapeDtypeStruct((NDP, H), jnp.float32)
    fcol = jax.ShapeDtypeStruct((NDP, 1), jnp.float32)
    return pl.pallas_call(
        _k1a_body,
        grid=(nblk,),
        in_specs=[row, row, full, full, full, vec, vec, vec, vec, vec],
        out_specs=[row, row, row, col, col, col, col, col],
        out_shape=[fmat, fmat, fmat, fcol, fcol, fcol, fcol, fcol],
    )(xd, xp, wdd, wpd, wpc, asdd, addd, aspd, adpd, aspc)


def _k1b_body(xc, wpc, adpc, lin, ae, sdpc_o, cvec_o):
    m = jnp.dot(xc[...], wpc[...], preferred_element_type=jnp.float32)
    sdpc_o[...] = (m * adpc[...]).sum(axis=1, keepdims=True)
    c = jnp.sum(lin[...] * ae[...])
    cvec_o[...] = jnp.full((1, H), c, jnp.float32)


def _k1b(xc, wpc, adpc, lin, ae):
    blk = pl.BlockSpec((NCP, H), lambda: (0, 0))
    full = pl.BlockSpec((H, H), lambda: (0, 0))
    vec = pl.BlockSpec((1, H), lambda: (0, 0))
    col = pl.BlockSpec((NCP, 1), lambda: (0, 0))
    return pl.pallas_call(
        _k1b_body,
        grid=(),
        in_specs=[blk, full, vec, vec, vec],
        out_specs=[col, vec],
        out_shape=[jax.ShapeDtypeStruct((NCP, 1), jnp.float32),
                   jax.ShapeDtypeStruct((1, H), jnp.float32)],
    )(xc, wpc, adpc, lin, ae)


def _bcast_lane(v, i):
    idx = jnp.full((16,), i, jnp.int32)
    return lax.gather(
        v, idx[:, None],
        dimension_numbers=lax.GatherDimensionNumbers(
            offset_dims=(), collapsed_slice_dims=(0,), start_index_map=(0,)),
        slice_sizes=(1,), mode=lax.GatherScatterMode.PROMISE_IN_BOUNDS)


def _s1_body(src_pd, dst_pd, src_dd, dst_dd, ea_dd, src_pc, dst_pc,
             hdd, hpd, hpc, ssdd, sddd, sspd, sdpd, sspc, sdpc, cvec_h,
             q1, q2, qc,
             g_dd1, g_dd2, g_pd1, g_pd2, g_pc,
             out_sh, outc_sh, den_sh, denc_sh, cnt_sh, sat_sh,
             tabA, tabB, idxs, idxd, lidx, qidx, eab, exb, onesb,
             cnb, sab, rows, cv16, gsem):
    cid = lax.axis_index("c")
    sid = lax.axis_index("s")
    zf = jnp.zeros((16,), jnp.float32)

    pltpu.sync_copy(cvec_h, cv16)

    def _zrow(r, _):
        for j in range(8):
            rows[r, pl.ds(16 * j, 16)] = zf
        return 0
    lax.fori_loop(0, K, _zrow, 0)

    def _zvec(g, _):
        exb[pl.ds(16 * g, 16)] = zf
        onesb[pl.ds(16 * g, 16)] = jnp.ones((16,), jnp.float32)
        return 0
    lax.fori_loop(0, K // 16, _zvec, 0)

    nb = NDP // NT
    nc = NCP // NT
    rb = sid * nb
    rc = sid * nc
    def _zero_shared(part, _):
        off = rb + part * K
        pltpu.sync_copy(rows, out_sh.at[pl.ds(off, K)])
        pltpu.sync_copy(exb, den_sh.at[pl.ds(off, K)])
        pltpu.sync_copy(exb, cnt_sh.at[pl.ds(off, K)])
        pltpu.sync_copy(exb, sat_sh.at[pl.ds(off, K)])
        return 0
    lax.fori_loop(0, nb // K, _zero_shared, 0)
    pltpu.sync_copy(rows.at[pl.ds(0, nc)], outc_sh.at[pl.ds(rc, nc)])
    pltpu.sync_copy(exb.at[pl.ds(0, nc)], denc_sh.at[pl.ds(rc, nc)])
    plsc.subcore_barrier()

    cv = cv16[...]

    def _scale_rows(nrow):
        def body(g, _):
            ev = exb[pl.ds(16 * g, 16)]
            for i in range(16):
                w = _bcast_lane(ev, i)
                e = 16 * g + i
                for j in range(8):
                    rows[e, pl.ds(16 * j, 16)] = rows[e, pl.ds(16 * j, 16)] * w
            return 0
        lax.fori_loop(0, nrow // 16, body, 0)

    def _edge_pass(etot, src_h, dst_h, hs_h, o_sh, d_sh, with_ea, with_cnt):
        per_tile = etot // NT
        base = sid * per_tile

        def chunk(ci, _):
            off = base + ci * K
            pltpu.sync_copy(src_h.at[pl.ds(off, K)], idxs)
            pltpu.sync_copy(dst_h.at[pl.ds(off, K)], idxd)
            if with_ea:
                pltpu.sync_copy(ea_dd.at[pl.ds(off, K)], eab)
            gat = pltpu.async_copy(hs_h.at[idxs], rows, gsem)

            def grp(g, _):
                sv = idxs[pl.ds(16 * g, 16)]
                dv = idxd[pl.ds(16 * g, 16)]
                a = plsc.load_gather(tabA, [sv]) + plsc.load_gather(tabB, [dv])
                if with_ea:
                    a = a + cv * eab[pl.ds(16 * g, 16)]
                a = jnp.where(a >= 0., a, jnp.float32(0.2) * a)
                exb[pl.ds(16 * g, 16)] = jnp.exp(a)
                return 0
            lax.fori_loop(0, K // 16, grp, 0)

            gat.wait()
            _scale_rows(K)
            pltpu.sync_copy(rows, o_sh.at[idxd], add=True)
            pltpu.sync_copy(exb, d_sh.at[idxd], add=True)
            if with_cnt:
                pltpu.sync_copy(onesb, cnt_sh.at[idxd], add=True)
                pltpu.sync_copy(eab, sat_sh.at[idxd], add=True)
            return 0
        lax.fori_loop(0, per_tile // K, chunk, 0)

    @pl.when(cid == 0)
    def _():
        pltpu.sync_copy(sspd, tabA)
        pltpu.sync_copy(sdpd, tabB)
        _edge_pass(EPD, src_pd, dst_pd, hpd, out_sh, den_sh, False, False)

    @pl.when(cid == 1)
    def _():
        pltpu.sync_copy(ssdd, tabA)
        pltpu.sync_copy(sddd, tabB)
        _edge_pass(EDD, src_dd, dst_dd, hdd, out_sh, den_sh, True, True)

    plsc.subcore_barrier()

    @pl.when(cid == 1)
    def _():
        def lchunk(ci, _):
            start = sid * nb + ci * KL
            pltpu.sync_copy(cnt_sh.at[pl.ds(start, KL)], cnb)
            pltpu.sync_copy(sat_sh.at[pl.ds(start, KL)], sab)
            pltpu.sync_copy(hdd.at[pl.ds(start, KL)], rows.at[pl.ds(0, KL)])

            def grp(g, _):
                iv = jnp.full((16,), start + 16 * g, jnp.int32) + lax.iota(jnp.int32, 16)
                lidx[pl.ds(16 * g, 16)] = iv
                cnt = cnb[pl.ds(16 * g, 16)]
                sat = sab[pl.ds(16 * g, 16)]
                mean = sat / jnp.maximum(cnt, jnp.float32(1.0))
                a = (tabA[pl.ds(start + 16 * g, 16)]
                     + tabB[pl.ds(start + 16 * g, 16)] + cv * mean)
                a = jnp.where(a >= 0., a, jnp.float32(0.2) * a)
                exb[pl.ds(16 * g, 16)] = jnp.exp(a)
                return 0
            lax.fori_loop(0, KL // 16, grp, 0)

            _scale_rows(KL)
            pltpu.sync_copy(rows.at[pl.ds(0, KL)], out_sh.at[lidx], add=True)
            pltpu.sync_copy(exb.at[pl.ds(0, KL)], den_sh.at[lidx], add=True)
            return 0
        lax.fori_loop(0, nb // KL, lchunk, 0)

        pltpu.sync_copy(sspc, tabA)
        pltpu.sync_copy(sdpc, tabB.at[pl.ds(0, NCP)])
        _edge_pass(EPC, src_pc, dst_pc, hpc, outc_sh, denc_sh, False, False)

    plsc.subcore_barrier()

    pltpu.sync_copy(den_sh, tabA)

    nq = B // NT

    def _qstream(qh, o_sh, dtab, gout):
        def qchunk(ci, _):
            qoff = sid * nq + ci * KQ
            pltpu.sync_copy(qh.at[pl.ds(qoff, KQ)], qidx)
            pltpu.sync_copy(o_sh.at[qidx], rows.at[pl.ds(0, KQ)])

            def grp(g, _):
                qv = qidx[pl.ds(16 * g, 16)]
                dv = plsc.load_gather(dtab, [qv])
                exb[pl.ds(16 * g, 16)] = (jnp.float32(1.0)
                                          / (dv + jnp.float32(1e-16)))
                return 0
            lax.fori_loop(0, KQ // 16, grp, 0)

            _scale_rows(KQ)
            pltpu.sync_copy(rows.at[pl.ds(0, KQ)], gout.at[pl.ds(qoff, KQ)])
            return 0
        lax.fori_loop(0, nq // KQ, qchunk, 0)

    @pl.when(cid == 0)
    def _():
        _qstream(q1, out_sh, tabA, g_pd1)
        _qstream(q2, out_sh, tabA, g_pd2)

    @pl.when(cid == 1)
    def _():
        _qstream(q1, out_sh, tabA, g_dd1)
        _qstream(q2, out_sh, tabA, g_dd2)
        pltpu.sync_copy(denc_sh, tabB.at[pl.ds(0, NCP)])
        _qstream(qc, outc_sh, tabB, g_pc)


def _s1(src_pd, dst_pd, src_dd, dst_dd, ea_dd, src_pc, dst_pc,
        hdd, hpd, hpc, ssdd, sddd, sspd, sdpd, sspc, sdpc, cvec,
        q1, q2, qc):
    fg = jax.ShapeDtypeStruct((B, H), jnp.float32)
    mesh = plsc.VectorSubcoreMesh(core_axis_name="c", subcore_axis_name="s")
    fn = pl.kernel(
        _s1_body,
        out_type=[fg, fg, fg, fg, fg],
        mesh=mesh,
        compiler_params=pltpu.CompilerParams(needs_layout_passes=False),
        scratch_types=[
            pltpu.VMEM_SHARED((NDP, H), jnp.float32),
            pltpu.VMEM_SHARED((NCP, H), jnp.float32),
            pltpu.VMEM_SHARED((NDP,), jnp.float32),
            pltpu.VMEM_SHARED((NCP,), jnp.float32),
            pltpu.VMEM_SHARED((NDP,), jnp.float32),
            pltpu.VMEM_SHARED((NDP,), jnp.float32),
            pltpu.VMEM((NDP,), jnp.float32),
            pltpu.VMEM((NDP,), jnp.float32),
            pltpu.VMEM((K,), jnp.int32),
            pltpu.VMEM((K,), jnp.int32),
            pltpu.VMEM((KL,), jnp.int32),
            pltpu.VMEM((KQ,), jnp.int32),
            pltpu.VMEM((K,), jnp.float32),
            pltpu.VMEM((K,), jnp.float32),
            pltpu.VMEM((K,), jnp.float32),
            pltpu.VMEM((KL,), jnp.float32),
            pltpu.VMEM((KL,), jnp.float32),
            pltpu.VMEM((K, H), jnp.float32),
            pltpu.VMEM((16,), jnp.float32),
            pltpu.SemaphoreType.DMA,
        ],
    )
    return fn(src_pd, dst_pd, src_dd, dst_dd, ea_dd, src_pc, dst_pc,
              hdd, hpd, hpc, ssdd, sddd, sspd, sdpd, sspc, sdpc, cvec,
              q1, q2, qc)


def _k2_body(gdd1, gpd1, gdd2, gpd2, gpc, bdd, bpd, bpc,
             w1, b1, w2, b2, w3, b3, out_o):
    def norml2(v):
        n = jnp.sqrt((v * v).sum(axis=1, keepdims=True))
        return v / jnp.maximum(n, jnp.float32(1e-12))

    bd = bdd[...] + bpd[...]
    h1 = norml2(jax.nn.relu(gdd1[...] + gpd1[...] + bd))
    h2 = norml2(jax.nn.relu(gdd2[...] + gpd2[...] + bd))
    h3 = norml2(jax.nn.relu(gpc[...] + bpc[...]))
    hid = jnp.concatenate([h1, h2, h3], axis=1)
    z = jax.nn.relu(jnp.dot(hid, w1[...], preferred_element_type=jnp.float32)
                    + b1[...])
    z = jax.nn.relu(jnp.dot(z, w2[...], preferred_element_type=jnp.float32)
                    + b2[...])
    out_o[...] = (jnp.dot(z, w3[...], preferred_element_type=jnp.float32)
                  + b3[...])


def _k2(gdd1, gpd1, gdd2, gpd2, gpc, bdd, bpd, bpc, w1, b1, w2, b2, w3p, b3p):
    nblk = B // 512
    row = pl.BlockSpec((512, H), lambda i: (i, 0))
    vec = pl.BlockSpec((1, H), lambda i: (0, 0))

    def fullspec(a):
        return pl.BlockSpec(a.shape, lambda i: tuple(0 for _ in a.shape))

    return pl.pallas_call(
        _k2_body,
        grid=(nblk,),
        in_specs=[row, row, row, row, row, vec, vec, vec,
                  fullspec(w1), fullspec(b1), fullspec(w2), fullspec(b2),
                  fullspec(w3p), fullspec(b3p)],
        out_specs=row,
        out_shape=jax.ShapeDtypeStruct((B, H), jnp.float32),
    )(gdd1, gpd1, gdd2, gpd2, gpc, bdd, bpd, bpc, w1, b1, w2, b2, w3p, b3p)


def kernel(x_drug, x_protein, x_cell, edge_index_dd, edge_attr_dd,
           edge_src_pd, edge_dst_pd, edge_src_pc, edge_dst_pc,
           drug1, drug2, cell, drug_emb, protein_emb, cell_emb,
           W_dd, att_src_dd, att_dst_dd, lin_edge_dd, att_edge_dd, bias_dd,
           W_pd, att_src_pd, att_dst_pd, bias_pd,
           W_pc, att_src_pc, att_dst_pc, bias_pc,
           W1, b1, W2, b2, W3, b3):
    f32 = jnp.float32
    nd = drug_emb.shape[0]
    nc = cell_emb.shape[0]
    xd = jnp.pad(drug_emb, ((0, NDP - nd), (0, 0)))
    xp = jnp.pad(protein_emb, ((0, NDP - protein_emb.shape[0]), (0, 0)))
    xc = jnp.pad(cell_emb, ((0, NCP - nc), (0, 0)))

    r1 = lambda v: v.reshape(1, H)
    hdd, hpd, hpc, ssdd, sddd, sspd, sdpd, sspc = _k1a(
        xd, xp, W_dd, W_pd, W_pc, r1(att_src_dd), r1(att_dst_dd),
        r1(att_src_pd), r1(att_dst_pd), r1(att_src_pc))
    sdpc, cvec = _k1b(xc, W_pc, r1(att_dst_pc), lin_edge_dd.reshape(1, H),
                      att_edge_dd.reshape(1, H))

    i32 = jnp.int32
    src_dd = edge_index_dd[0].astype(i32)
    dst_dd = edge_index_dd[1].astype(i32)
    ea_dd = edge_attr_dd[:, 0].astype(f32)

    g_dd1, g_dd2, g_pd1, g_pd2, g_pc = _s1(
        edge_src_pd.astype(i32), edge_dst_pd.astype(i32),
        src_dd, dst_dd, ea_dd,
        edge_src_pc.astype(i32), edge_dst_pc.astype(i32),
        hdd, hpd, hpc,
        ssdd.reshape(NDP), sddd.reshape(NDP), sspd.reshape(NDP),
        sdpd.reshape(NDP), sspc.reshape(NDP), sdpc.reshape(NCP),
        cvec[0, :16],
        drug1.astype(i32), drug2.astype(i32), cell.astype(i32))

    w3p = jnp.pad(W3, ((0, 0), (0, H - W3.shape[1])))
    b3p = jnp.pad(b3, (0, H - b3.shape[0])).reshape(1, H)
    out = _k2(g_dd1, g_pd1, g_dd2, g_pd2, g_pc,
              r1(bias_dd), r1(bias_pd), r1(bias_pc),
              W1, b1.reshape(1, W1.shape[1]), W2, b2.reshape(1, W2.shape[1]),
              w3p, b3p)
    return out[:, :W3.shape[1]]
